# baseline (speedup 1.0000x reference)
import math
import numpy as np
import jax
import jax.numpy as jnp
from jax import lax
from jax.experimental import pallas as pl
from jax.experimental.pallas import tpu as pltpu
from jax.experimental.pallas import tpu_sc as plsc

NMOL = 128
A = 32
G = 24
G2 = 32
O = 20
L = 4
NSP = 4
RCUT = 6.0
ETA = 1.2
NW = 32
MPW = NMOL // NW
ROW = 960
MROW = A * ROW

_COMPS = []
_LIDX = []
for _i in range(L):
    for _k in range(_i + 1):
        for _m in range(_i - _k + 1):
            _COMPS.append((_i - _k - _m, _m, _k))
            _LIDX.append(_i)
_CHANS = [(e, e) for e in range(NSP)] + [
    (a, b) for a in range(NSP) for b in range(a + 1, NSP)
]
NCH = len(_CHANS)

_COSC = [((-1.0) ** k) / math.factorial(2 * k) for k in range(10)]


def _rsqrt16(r2):
    i = lax.bitcast_convert_type(r2, jnp.int32)
    i = jnp.int32(0x5F3759DF) - lax.shift_right_logical(i, 1)
    y = lax.bitcast_convert_type(i, jnp.float32)
    h = 0.5 * r2
    for _ in range(3):
        y = y * (1.5 - h * y * y)
    return y


def _cospi16(z):
    u = z * z
    acc = jnp.full((16,), _COSC[9], jnp.float32)
    for k in range(8, -1, -1):
        acc = acc * u + _COSC[k]
    return acc


def _splat(v, n=16):
    return jnp.full((n,), v, jnp.int32)


def _sc_body(xs_hbm, xsp_hbm, meta_hbm, cst_hbm, out_hbm,
             xs_vm, xsp_vm, meta_vm, cst_vm, angT_vm, ts_vm, stg_vm,
             out_vm, sem):
    wid = lax.axis_index("s") * 2 + lax.axis_index("c")

    pltpu.sync_copy(cst_hbm, cst_vm)
    mu_lo = cst_vm[pl.ds(0, 16)]
    mu_hi = cst_vm[pl.ds(16, 16)]
    wv_lo = cst_vm[pl.ds(32, 16)]
    wv_hi = cst_vm[pl.ds(48, 16)]
    iv_lo = cst_vm[pl.ds(64, 16)]
    iv_hi = cst_vm[pl.ds(80, 16)]
    fv = [cst_vm[pl.ds(96 + 16 * q, 16)] for q in range(3)]
    lane = lax.iota(jnp.int32, 16)
    lane32 = lane * G2

    def per_mol(m, _):
        b = wid * MPW + m
        pltpu.sync_copy(xs_hbm.at[b], xs_vm)
        pltpu.sync_copy(xsp_hbm.at[b], xsp_vm)
        pltpu.sync_copy(meta_hbm.at[b], meta_vm)

        def chunk_body(c, _):
            toff = pl.multiple_of((c & 1) * 16, 16)
            c16 = pl.multiple_of(c * 16, 16)
            rowbase = lane32 + c * (16 * G2)
            xi = [xsp_vm[pl.ds(pl.multiple_of(cc * (64 * 16) + c16, 16), 16)]
                  for cc in range(3)]
            dx = xs_vm[pl.ds(pl.multiple_of(0 * A + toff, 16), 16)] - xi[0]
            dy = xs_vm[pl.ds(pl.multiple_of(1 * A + toff, 16), 16)] - xi[1]
            dz = xs_vm[pl.ds(pl.multiple_of(2 * A + toff, 16), 16)] - xi[2]
            r2 = dx * dx + dy * dy + dz * dz
            valid = (r2 > 1e-10) & (r2 < RCUT * RCUT)
            r2s = jnp.where(r2 > 1e-10, r2, 1.0)
            rinv = _rsqrt16(r2s)
            r = r2s * rinv
            fcut = 0.5 * (_cospi16(r * (np.pi / RCUT)) + 1.0)
            fcut = jnp.where(valid, fcut, 0.0)
            ux = dx * rinv
            uy = dy * rinv
            uz = dz * rinv
            px = [None, ux, ux * ux, None]
            py = [None, uy, uy * uy, None]
            pz = [None, uz, uz * uz, None]
            px[3] = px[2] * ux
            py[3] = py[2] * uy
            pz[3] = pz[2] * uz
            for o in range(O):
                n, mm, k = _COMPS[o]
                t = None
                for pw, p in ((px, n), (py, mm), (pz, k)):
                    if p > 0:
                        t = pw[p] if t is None else t * pw[p]
                wo = wv_lo[o] if o < 16 else wv_hi[o - 16]
                af = fcut * wo if t is None else t * (fcut * wo)
                plsc.store_scatter(angT_vm, [rowbase + o], af)
            plsc.store_scatter(angT_vm, [rowbase + O], r)
            return 0

        lax.fori_loop(0, 2 * A, chunk_body, 0)

        stv = meta_vm[pl.ds(0, 16)]

        def atom_body(i, _):
            base = i * (A * G2)

            zv = jnp.zeros((16,), jnp.float32)
            for e in range(NSP):
                t0 = stv[e]
                t1 = stv[e + 1]
                eb = e * (O * G2)

                init = tuple(zv for _ in range(2 * O))

                @plsc.parallel_loop(t0, t1, unroll=2, carry=init)
                def pair_body(t, acc):
                    roff = pl.multiple_of(base + t * G2, 16)
                    row_lo = angT_vm[pl.ds(roff, 16)]
                    row_hi = angT_vm[pl.ds(roff + 16, 16)]
                    rr = row_hi[O - 16]
                    d_lo = mu_lo - rr
                    d_hi = mu_hi - rr
                    rad_lo = jnp.exp(d_lo * d_lo * (-ETA))
                    rad_hi = jnp.exp(d_hi * d_hi * (-ETA))
                    new = []
                    for o in range(O):
                        ao = row_lo[o] if o < 16 else row_hi[o - 16]
                        new.append(acc[2 * o] + ao * rad_lo)
                        new.append(acc[2 * o + 1] + ao * rad_hi)
                    return tuple(new)

                accf = pair_body
                for o in range(O):
                    ts_vm[pl.ds(eb + o * G2, 16)] = accf[2 * o]
                    ts_vm[pl.ds(eb + o * G2 + 16, 16)] = accf[2 * o + 1]

            for l in range(L):
                osub = [o for o in range(O) if _LIDX[o] == l]
                acc = [[None, None] for _ in range(NCH)]
                for o in osub:
                    invf = iv_lo[o] if o < 16 else iv_hi[o - 16]
                    for h in range(2):
                        te = [ts_vm[pl.ds(e * O * G2 + o * G2 + h * 16, 16)]
                              for e in range(NSP)]
                        qe = [te[e] * invf for e in range(NSP)]
                        for ci, (a, bb) in enumerate(_CHANS):
                            term = qe[a] * te[bb]
                            if acc[ci][h] is None:
                                acc[ci][h] = term
                            else:
                                acc[ci][h] = acc[ci][h] + term
                for ci in range(NCH):
                    q = ci * L + l
                    fac = fv[q // 16][q % 16]
                    sb = l * NCH * G2 + ci * G2
                    stg_vm[pl.ds(sb, 16)] = acc[ci][0] * fac
                    stg_vm[pl.ds(sb + 16, 16)] = acc[ci][1] * fac

            obase = i * ROW
            for ci in range(NCH):
                for l in range(L):
                    off = pl.multiple_of(obase + ci * (L * G) + l * G, 8)
                    sb = l * NCH * G2 + ci * G2
                    out_vm[pl.ds(off, 16)] = stg_vm[pl.ds(sb, 16)]
                    out_vm[pl.ds(off + 16, 16)] = stg_vm[pl.ds(sb + 16, 16)]
            return 0

        lax.fori_loop(0, A, atom_body, 0)
        pltpu.sync_copy(out_vm.at[pl.ds(0, MROW)], out_hbm.at[b])
        return 0

    lax.fori_loop(0, MPW, per_mol, 0)


def kernel(X, Z, atomIDs, molIDs, atom_counts, species, offset,
           orbital_components, orbital_weights, orbital_indexes,
           inv_factors_orb, lchannel_weights):
    f32 = jnp.float32
    nmol = X.shape[0]
    eid = jnp.argmax(Z[..., None] == species[None, None, :], axis=-1)
    eid = eid.astype(jnp.int32)
    onehot = jax.nn.one_hot(eid, NSP, dtype=f32)
    counts = jnp.sum(onehot, axis=1)
    pfx = jnp.concatenate(
        [jnp.zeros((nmol, 1), f32), jnp.cumsum(counts, axis=1)[:, :NSP - 1]],
        axis=1)
    within = jnp.cumsum(onehot, axis=1) - onehot
    pos = jnp.sum((within + pfx[:, None, :]) * onehot, axis=-1)
    perm1h = jax.nn.one_hot(pos.astype(jnp.int32), A, dtype=f32)
    Xs = jnp.einsum('bjp,bjc->bpc', perm1h, X,
                    precision=lax.Precision.HIGHEST)
    Xs_t = jnp.concatenate(
        [jnp.transpose(Xs, (0, 2, 1)), jnp.zeros((nmol, 1, A), f32)], axis=1)
    Xs_f = Xs_t.reshape(nmol, NSP * A)
    Xt_t3 = jnp.transpose(X, (0, 2, 1))
    Xsp = jnp.broadcast_to(Xt_t3[:, :, :, None, None],
                           (nmol, 3, A, 2, 16)).reshape(nmol, 3 * 64 * 16)
    starts = jnp.concatenate(
        [jnp.zeros((nmol, 1), f32), jnp.cumsum(counts, axis=1)],
        axis=1).astype(jnp.int32)
    meta = jnp.concatenate(
        [starts, jnp.zeros((nmol, 59), jnp.int32)], axis=1)

    mu = jnp.concatenate([offset.astype(f32), jnp.full((G2 - G,), 1e9, f32)])
    wv = jnp.concatenate([orbital_weights.astype(f32),
                          jnp.zeros((G2 - O,), f32)])
    iv = jnp.concatenate([inv_factors_orb.astype(f32),
                          jnp.zeros((G2 - O,), f32)])
    facs = []
    for (a, b) in _CHANS:
        f = 1.0 if a == b else 2.0
        facs.append(lchannel_weights.astype(f32) * f)
    fac = jnp.concatenate(facs)
    cst = jnp.concatenate([mu, wv, iv, fac, jnp.zeros((8,), f32)])

    mesh = plsc.VectorSubcoreMesh(core_axis_name="c", subcore_axis_name="s",
                                  num_cores=2, num_subcores=16)
    run = pl.kernel(
        _sc_body,
        mesh=mesh,
        compiler_params=pltpu.CompilerParams(needs_layout_passes=False),
        out_type=jax.ShapeDtypeStruct((nmol, MROW), f32),
        scratch_types=[
            pltpu.VMEM((NSP * A,), f32),
            pltpu.VMEM((3 * 64 * 16,), f32),
            pltpu.VMEM((64,), jnp.int32),
            pltpu.VMEM((144,), f32),
            pltpu.VMEM((A * A * G2,), f32),
            pltpu.VMEM((NSP * O * G2,), f32),
            pltpu.VMEM((L * NCH * G2,), f32),
            pltpu.VMEM((MROW + 16,), f32),
            pltpu.SemaphoreType.DMA,
        ],
    )
    out = run(Xs_f, Xsp, meta, cst)
    return out.reshape(nmol, A, ROW)


# speedup vs baseline: 1.2177x; 1.2177x over previous
import math
import numpy as np
import jax
import jax.numpy as jnp
from jax import lax
from jax.experimental import pallas as pl
from jax.experimental.pallas import tpu as pltpu
from jax.experimental.pallas import tpu_sc as plsc

NMOL = 128
A = 32
G = 24
G2 = 32
O = 20
L = 4
NSP = 4
RCUT = 6.0
ETA = 1.2
NW = 32
MPW = NMOL // NW
ROW = 960
MROW = A * ROW

_COMPS = []
_LIDX = []
for _i in range(L):
    for _k in range(_i + 1):
        for _m in range(_i - _k + 1):
            _COMPS.append((_i - _k - _m, _m, _k))
            _LIDX.append(_i)
_CHANS = [(e, e) for e in range(NSP)] + [
    (a, b) for a in range(NSP) for b in range(a + 1, NSP)
]
NCH = len(_CHANS)

_COSC = [((-1.0) ** k) / math.factorial(2 * k) for k in range(10)]


def _rsqrt16(r2):
    i = lax.bitcast_convert_type(r2, jnp.int32)
    i = jnp.int32(0x5F3759DF) - lax.shift_right_logical(i, 1)
    y = lax.bitcast_convert_type(i, jnp.float32)
    h = 0.5 * r2
    for _ in range(3):
        y = y * (1.5 - h * y * y)
    return y


def _cospi16(z):
    u = z * z
    acc = jnp.full((16,), _COSC[9], jnp.float32)
    for k in range(8, -1, -1):
        acc = acc * u + _COSC[k]
    return acc


def _splat(v, n=16):
    return jnp.full((n,), v, jnp.int32)


def _sc_body(xs_hbm, xsp_hbm, meta_hbm, cst_hbm, out_hbm,
             xs_vm, xsp_vm, meta_vm, cst_vm, angT_vm, ts_vm, stg_vm,
             out_vm, sem):
    wid = lax.axis_index("s") * 2 + lax.axis_index("c")

    pltpu.sync_copy(cst_hbm, cst_vm)
    mu_lo = cst_vm[pl.ds(0, 16)]
    mu_hi = cst_vm[pl.ds(16, 16)]
    wv_lo = cst_vm[pl.ds(32, 16)]
    wv_hi = cst_vm[pl.ds(48, 16)]
    iv_lo = cst_vm[pl.ds(64, 16)]
    iv_hi = cst_vm[pl.ds(80, 16)]
    fv = [cst_vm[pl.ds(96 + 16 * q, 16)] for q in range(3)]
    lane = lax.iota(jnp.int32, 16)
    lane32 = lane * G2

    def per_mol(m, _):
        b = wid * MPW + m
        pltpu.sync_copy(xs_hbm.at[b], xs_vm)
        pltpu.sync_copy(xsp_hbm.at[b], xsp_vm)
        pltpu.sync_copy(meta_hbm.at[b], meta_vm)

        def chunk_body(c, _):
            toff = pl.multiple_of((c & 1) * 16, 16)
            c16 = pl.multiple_of(c * 16, 16)
            rowbase = lane32 + c * (16 * G2)
            xi = [xsp_vm[pl.ds(pl.multiple_of(cc * (64 * 16) + c16, 16), 16)]
                  for cc in range(3)]
            dx = xs_vm[pl.ds(pl.multiple_of(0 * A + toff, 16), 16)] - xi[0]
            dy = xs_vm[pl.ds(pl.multiple_of(1 * A + toff, 16), 16)] - xi[1]
            dz = xs_vm[pl.ds(pl.multiple_of(2 * A + toff, 16), 16)] - xi[2]
            r2 = dx * dx + dy * dy + dz * dz
            valid = (r2 > 1e-10) & (r2 < RCUT * RCUT)
            r2s = jnp.where(r2 > 1e-10, r2, 1.0)
            rinv = _rsqrt16(r2s)
            r = r2s * rinv
            fcut = 0.5 * (_cospi16(r * (np.pi / RCUT)) + 1.0)
            fcut = jnp.where(valid, fcut, 0.0)
            ux = dx * rinv
            uy = dy * rinv
            uz = dz * rinv
            px = [None, ux, ux * ux, None]
            py = [None, uy, uy * uy, None]
            pz = [None, uz, uz * uz, None]
            px[3] = px[2] * ux
            py[3] = py[2] * uy
            pz[3] = pz[2] * uz
            for o in range(O):
                n, mm, k = _COMPS[o]
                t = None
                for pw, p in ((px, n), (py, mm), (pz, k)):
                    if p > 0:
                        t = pw[p] if t is None else t * pw[p]
                wo = wv_lo[o] if o < 16 else wv_hi[o - 16]
                af = fcut * wo if t is None else t * (fcut * wo)
                plsc.store_scatter(angT_vm, [rowbase + o], af)
            plsc.store_scatter(angT_vm, [rowbase + O], r)
            return 0

        lax.fori_loop(0, 2 * A, chunk_body, 0)

        stv = meta_vm[pl.ds(0, 16)]

        def atom_body(i, _):
            base = i * (A * G2)

            zv = jnp.zeros((16,), jnp.float32)
            for e in range(NSP):
                t0 = stv[e]
                t1 = stv[e + 1]
                eb = e * (O * G2)

                for half in range(2):
                    olo = half * (O // 2)
                    init = tuple(zv for _ in range(O))

                    @plsc.parallel_loop(t0, t1, unroll=2, carry=init)
                    def pair_body(t, acc, olo=olo):
                        roff = pl.multiple_of(base + t * G2, 16)
                        row_lo = angT_vm[pl.ds(roff, 16)]
                        row_hi = angT_vm[pl.ds(roff + 16, 16)]
                        rr = row_hi[O - 16]
                        d_lo = mu_lo - rr
                        d_hi = mu_hi - rr
                        rad_lo = jnp.exp(d_lo * d_lo * (-ETA))
                        rad_hi = jnp.exp(d_hi * d_hi * (-ETA))
                        new = []
                        for q in range(O // 2):
                            o = olo + q
                            ao = row_lo[o] if o < 16 else row_hi[o - 16]
                            new.append(acc[2 * q] + ao * rad_lo)
                            new.append(acc[2 * q + 1] + ao * rad_hi)
                        return tuple(new)

                    accf = pair_body
                    for q in range(O // 2):
                        o = olo + q
                        ts_vm[pl.ds(eb + o * G2, 16)] = accf[2 * q]
                        ts_vm[pl.ds(eb + o * G2 + 16, 16)] = accf[2 * q + 1]

            for l in range(L):
                osub = [o for o in range(O) if _LIDX[o] == l]
                acc = [[None, None] for _ in range(NCH)]
                for o in osub:
                    invf = iv_lo[o] if o < 16 else iv_hi[o - 16]
                    for h in range(2):
                        te = [ts_vm[pl.ds(e * O * G2 + o * G2 + h * 16, 16)]
                              for e in range(NSP)]
                        qe = [te[e] * invf for e in range(NSP)]
                        for ci, (a, bb) in enumerate(_CHANS):
                            term = qe[a] * te[bb]
                            if acc[ci][h] is None:
                                acc[ci][h] = term
                            else:
                                acc[ci][h] = acc[ci][h] + term
                for ci in range(NCH):
                    q = ci * L + l
                    fac = fv[q // 16][q % 16]
                    sb = l * NCH * G2 + ci * G2
                    stg_vm[pl.ds(sb, 16)] = acc[ci][0] * fac
                    stg_vm[pl.ds(sb + 16, 16)] = acc[ci][1] * fac

            obase = i * ROW
            for ci in range(NCH):
                for l in range(L):
                    off = pl.multiple_of(obase + ci * (L * G) + l * G, 8)
                    sb = l * NCH * G2 + ci * G2
                    out_vm[pl.ds(off, 16)] = stg_vm[pl.ds(sb, 16)]
                    out_vm[pl.ds(off + 16, 16)] = stg_vm[pl.ds(sb + 16, 16)]
            return 0

        lax.fori_loop(0, A, atom_body, 0)
        pltpu.sync_copy(out_vm.at[pl.ds(0, MROW)], out_hbm.at[b])
        return 0

    lax.fori_loop(0, MPW, per_mol, 0)


def kernel(X, Z, atomIDs, molIDs, atom_counts, species, offset,
           orbital_components, orbital_weights, orbital_indexes,
           inv_factors_orb, lchannel_weights):
    f32 = jnp.float32
    nmol = X.shape[0]
    eid = jnp.argmax(Z[..., None] == species[None, None, :], axis=-1)
    eid = eid.astype(jnp.int32)
    onehot = jax.nn.one_hot(eid, NSP, dtype=f32)
    counts = jnp.sum(onehot, axis=1)
    pfx = jnp.concatenate(
        [jnp.zeros((nmol, 1), f32), jnp.cumsum(counts, axis=1)[:, :NSP - 1]],
        axis=1)
    within = jnp.cumsum(onehot, axis=1) - onehot
    pos = jnp.sum((within + pfx[:, None, :]) * onehot, axis=-1)
    perm1h = jax.nn.one_hot(pos.astype(jnp.int32), A, dtype=f32)
    Xs = jnp.einsum('bjp,bjc->bpc', perm1h, X,
                    precision=lax.Precision.HIGHEST)
    Xs_t = jnp.concatenate(
        [jnp.transpose(Xs, (0, 2, 1)), jnp.zeros((nmol, 1, A), f32)], axis=1)
    Xs_f = Xs_t.reshape(nmol, NSP * A)
    Xt_t3 = jnp.transpose(X, (0, 2, 1))
    Xsp = jnp.broadcast_to(Xt_t3[:, :, :, None, None],
                           (nmol, 3, A, 2, 16)).reshape(nmol, 3 * 64 * 16)
    starts = jnp.concatenate(
        [jnp.zeros((nmol, 1), f32), jnp.cumsum(counts, axis=1)],
        axis=1).astype(jnp.int32)
    meta = jnp.concatenate(
        [starts, jnp.zeros((nmol, 59), jnp.int32)], axis=1)

    mu = jnp.concatenate([offset.astype(f32), jnp.full((G2 - G,), 1e9, f32)])
    wv = jnp.concatenate([orbital_weights.astype(f32),
                          jnp.zeros((G2 - O,), f32)])
    iv = jnp.concatenate([inv_factors_orb.astype(f32),
                          jnp.zeros((G2 - O,), f32)])
    facs = []
    for (a, b) in _CHANS:
        f = 1.0 if a == b else 2.0
        facs.append(lchannel_weights.astype(f32) * f)
    fac = jnp.concatenate(facs)
    cst = jnp.concatenate([mu, wv, iv, fac, jnp.zeros((8,), f32)])

    mesh = plsc.VectorSubcoreMesh(core_axis_name="c", subcore_axis_name="s",
                                  num_cores=2, num_subcores=16)
    run = pl.kernel(
        _sc_body,
        mesh=mesh,
        compiler_params=pltpu.CompilerParams(needs_layout_passes=False),
        out_type=jax.ShapeDtypeStruct((nmol, MROW), f32),
        scratch_types=[
            pltpu.VMEM((NSP * A,), f32),
            pltpu.VMEM((3 * 64 * 16,), f32),
            pltpu.VMEM((64,), jnp.int32),
            pltpu.VMEM((144,), f32),
            pltpu.VMEM((A * A * G2,), f32),
            pltpu.VMEM((NSP * O * G2,), f32),
            pltpu.VMEM((L * NCH * G2,), f32),
            pltpu.VMEM((MROW + 16,), f32),
            pltpu.SemaphoreType.DMA,
        ],
    )
    out = run(Xs_f, Xsp, meta, cst)
    return out.reshape(nmol, A, ROW)


# speedup vs baseline: 1.7414x; 1.4302x over previous
import math
import numpy as np
import jax
import jax.numpy as jnp
from jax import lax
from jax.experimental import pallas as pl
from jax.experimental.pallas import tpu as pltpu
from jax.experimental.pallas import tpu_sc as plsc

NMOL = 128
A = 32
G = 24
G2 = 32
O = 20
L = 4
NSP = 4
RCUT = 6.0
ETA = 1.2
NW = 32
MPW = NMOL // NW
ROW = 960
MROW = A * ROW

_COMPS = []
_LIDX = []
for _i in range(L):
    for _k in range(_i + 1):
        for _m in range(_i - _k + 1):
            _COMPS.append((_i - _k - _m, _m, _k))
            _LIDX.append(_i)
_CHANS = [(e, e) for e in range(NSP)] + [
    (a, b) for a in range(NSP) for b in range(a + 1, NSP)
]
NCH = len(_CHANS)

_COSC = [((-1.0) ** k) / math.factorial(2 * k) for k in range(10)]


def _rsqrt16(r2):
    i = lax.bitcast_convert_type(r2, jnp.int32)
    i = jnp.int32(0x5F3759DF) - lax.shift_right_logical(i, 1)
    y = lax.bitcast_convert_type(i, jnp.float32)
    h = 0.5 * r2
    for _ in range(3):
        y = y * (1.5 - h * y * y)
    return y


def _cospi16(z):
    u = z * z
    acc = jnp.full((16,), _COSC[9], jnp.float32)
    for k in range(8, -1, -1):
        acc = acc * u + _COSC[k]
    return acc


def _splat(v, n=16):
    return jnp.full((n,), v, jnp.int32)


def _sc_body(xs_hbm, xsp_hbm, st_hbm, cst_hbm, out_hbm,
             xs_vm, xsp_vm, st_vm, cst_vm, angT_vm, ts_vm, stg_vm,
             out_vm, sem):
    wid = lax.axis_index("s") * 2 + lax.axis_index("c")

    pltpu.sync_copy(cst_hbm, cst_vm)
    mu_lo = cst_vm[pl.ds(0, 16)]
    mu_hi = cst_vm[pl.ds(16, 16)]
    wv_lo = cst_vm[pl.ds(32, 16)]
    wv_hi = cst_vm[pl.ds(48, 16)]
    iv_lo = cst_vm[pl.ds(64, 16)]
    iv_hi = cst_vm[pl.ds(80, 16)]
    fv = [cst_vm[pl.ds(96 + 16 * q, 16)] for q in range(3)]
    lane = lax.iota(jnp.int32, 16)
    lane32 = lane * G2

    def per_mol(m, _):
        b = wid * MPW + m
        pltpu.sync_copy(xs_hbm.at[b], xs_vm)
        pltpu.sync_copy(xsp_hbm.at[b], xsp_vm)
        pltpu.sync_copy(st_hbm.at[b], st_vm)
        stv = st_vm[pl.ds(0, 16)]

        def chunk_body(c, _):
            toff = pl.multiple_of((c & 1) * 16, 16)
            c16 = pl.multiple_of(c * 16, 16)
            rowbase = lane32 + c * (16 * G2)
            xi = [xsp_vm[pl.ds(pl.multiple_of(cc * (64 * 16) + c16, 16), 16)]
                  for cc in range(3)]
            dx = xs_vm[pl.ds(pl.multiple_of(0 * A + toff, 16), 16)] - xi[0]
            dy = xs_vm[pl.ds(pl.multiple_of(1 * A + toff, 16), 16)] - xi[1]
            dz = xs_vm[pl.ds(pl.multiple_of(2 * A + toff, 16), 16)] - xi[2]
            r2 = dx * dx + dy * dy + dz * dz
            valid = (r2 > 1e-10) & (r2 < RCUT * RCUT)
            r2s = jnp.where(r2 > 1e-10, r2, 1.0)
            rinv = _rsqrt16(r2s)
            r = r2s * rinv
            fcut = 0.5 * (_cospi16(r * (np.pi / RCUT)) + 1.0)
            fcut = jnp.where(valid, fcut, 0.0)
            ux = dx * rinv
            uy = dy * rinv
            uz = dz * rinv
            px = [None, ux, ux * ux, None]
            py = [None, uy, uy * uy, None]
            pz = [None, uz, uz * uz, None]
            px[3] = px[2] * ux
            py[3] = py[2] * uy
            pz[3] = pz[2] * uz
            for o in range(O):
                n, mm, k = _COMPS[o]
                t = None
                for pw, p in ((px, n), (py, mm), (pz, k)):
                    if p > 0:
                        t = pw[p] if t is None else t * pw[p]
                wo = wv_lo[o] if o < 16 else wv_hi[o - 16]
                af = fcut * wo if t is None else t * (fcut * wo)
                plsc.store_scatter(angT_vm, [rowbase + o], af)
            plsc.store_scatter(angT_vm, [rowbase + O], r)
            return 0

        lax.fori_loop(0, 2 * A, chunk_body, 0)

        def atom_body(i, _):
            base = i * A

            for e in range(NSP):
                t0 = stv[e]
                t1 = stv[e + 1]

                init = tuple(jnp.zeros((16,), jnp.float32)
                             for _ in range(2 * O))

                @plsc.parallel_loop(t0, t1, unroll=1, carry=init)
                def pair_body(t, acc):
                    roff = pl.multiple_of((base + t) * G2, 16)
                    row_lo = angT_vm[pl.ds(roff, 16)]
                    row_hi = angT_vm[pl.ds(roff + 16, 16)]
                    rr = row_hi[O - 16]
                    d_lo = mu_lo - rr
                    d_hi = mu_hi - rr
                    rad_lo = jnp.exp(d_lo * d_lo * (-ETA))
                    rad_hi = jnp.exp(d_hi * d_hi * (-ETA))
                    new = []
                    for o in range(O):
                        ao = row_lo[o] if o < 16 else row_hi[o - 16]
                        new.append(acc[2 * o] + ao * rad_lo)
                        new.append(acc[2 * o + 1] + ao * rad_hi)
                    return tuple(new)

                accf = pair_body
                for o in range(O):
                    ts_vm[pl.ds(e * O * G2 + o * G2, 16)] = accf[2 * o]
                    ts_vm[pl.ds(e * O * G2 + o * G2 + 16, 16)] = accf[2 * o + 1]

            for l in range(L):
                osub = [o for o in range(O) if _LIDX[o] == l]
                acc = [[None, None] for _ in range(NCH)]
                for o in osub:
                    invf = iv_lo[o] if o < 16 else iv_hi[o - 16]
                    for h in range(2):
                        te = [ts_vm[pl.ds(e * O * G2 + o * G2 + h * 16, 16)]
                              for e in range(NSP)]
                        qe = [te[e] * invf for e in range(NSP)]
                        for ci, (a, bb) in enumerate(_CHANS):
                            term = qe[a] * te[bb]
                            if acc[ci][h] is None:
                                acc[ci][h] = term
                            else:
                                acc[ci][h] = acc[ci][h] + term
                for ci in range(NCH):
                    q = ci * L + l
                    fac = fv[q // 16][q % 16]
                    sb = l * NCH * G2 + ci * G2
                    stg_vm[pl.ds(sb, 16)] = acc[ci][0] * fac
                    stg_vm[pl.ds(sb + 16, 16)] = acc[ci][1] * fac

            obase = i * ROW
            for ci in range(NCH):
                for l in range(L):
                    off = pl.multiple_of(obase + ci * (L * G) + l * G, 8)
                    sb = l * NCH * G2 + ci * G2
                    out_vm[pl.ds(off, 16)] = stg_vm[pl.ds(sb, 16)]
                    out_vm[pl.ds(off + 16, 16)] = stg_vm[pl.ds(sb + 16, 16)]
            return 0

        lax.fori_loop(0, A, atom_body, 0)
        pltpu.sync_copy(out_vm.at[pl.ds(0, MROW)], out_hbm.at[b])
        return 0

    lax.fori_loop(0, MPW, per_mol, 0)


def kernel(X, Z, atomIDs, molIDs, atom_counts, species, offset,
           orbital_components, orbital_weights, orbital_indexes,
           inv_factors_orb, lchannel_weights):
    f32 = jnp.float32
    nmol = X.shape[0]
    eid = jnp.argmax(Z[..., None] == species[None, None, :], axis=-1)
    eid = eid.astype(jnp.int32)
    onehot = jax.nn.one_hot(eid, NSP, dtype=f32)
    counts = jnp.sum(onehot, axis=1)
    pfx = jnp.concatenate(
        [jnp.zeros((nmol, 1), f32), jnp.cumsum(counts, axis=1)[:, :NSP - 1]],
        axis=1)
    within = jnp.cumsum(onehot, axis=1) - onehot
    pos = jnp.sum((within + pfx[:, None, :]) * onehot, axis=-1)
    perm1h = jax.nn.one_hot(pos.astype(jnp.int32), A, dtype=f32)
    Xs = jnp.einsum('bjp,bjc->bpc', perm1h, X,
                    precision=lax.Precision.HIGHEST)
    Xs_t = jnp.concatenate(
        [jnp.transpose(Xs, (0, 2, 1)), jnp.zeros((nmol, 1, A), f32)], axis=1)
    Xs_f = Xs_t.reshape(nmol, NSP * A)
    Xt_t3 = jnp.transpose(X, (0, 2, 1))
    Xsp = jnp.broadcast_to(Xt_t3[:, :, :, None, None],
                           (nmol, 3, A, 2, 16)).reshape(nmol, 3 * 64 * 16)
    starts = jnp.concatenate(
        [jnp.zeros((nmol, 1), f32), jnp.cumsum(counts, axis=1)],
        axis=1).astype(jnp.int32)
    starts = jnp.concatenate(
        [starts, jnp.zeros((nmol, 11), jnp.int32)], axis=1)

    mu = jnp.concatenate([offset.astype(f32), jnp.full((G2 - G,), 1e9, f32)])
    wv = jnp.concatenate([orbital_weights.astype(f32),
                          jnp.zeros((G2 - O,), f32)])
    iv = jnp.concatenate([inv_factors_orb.astype(f32),
                          jnp.zeros((G2 - O,), f32)])
    facs = []
    for (a, b) in _CHANS:
        f = 1.0 if a == b else 2.0
        facs.append(lchannel_weights.astype(f32) * f)
    fac = jnp.concatenate(facs)
    cst = jnp.concatenate([mu, wv, iv, fac, jnp.zeros((8,), f32)])

    mesh = plsc.VectorSubcoreMesh(core_axis_name="c", subcore_axis_name="s",
                                  num_cores=2, num_subcores=16)
    run = pl.kernel(
        _sc_body,
        mesh=mesh,
        compiler_params=pltpu.CompilerParams(needs_layout_passes=False),
        out_type=jax.ShapeDtypeStruct((nmol, MROW), f32),
        scratch_types=[
            pltpu.VMEM((NSP * A,), f32),
            pltpu.VMEM((3 * 64 * 16,), f32),
            pltpu.VMEM((16,), jnp.int32),
            pltpu.VMEM((144,), f32),
            pltpu.VMEM((A * A * G2,), f32),
            pltpu.VMEM((NSP * O * G2,), f32),
            pltpu.VMEM((L * NCH * G2,), f32),
            pltpu.VMEM((MROW + 16,), f32),
            pltpu.SemaphoreType.DMA,
        ],
    )
    out = run(Xs_f, Xsp, starts, cst)
    return out.reshape(nmol, A, ROW)
